# baseline (speedup 1.0000x reference)
import functools

import jax
import jax.numpy as jnp
from jax import lax
from jax.experimental import pallas as pl
from jax.experimental.pallas import tpu as pltpu
from jax.experimental.pallas import tpu_sc as plsc

B = 4096
NODE_ID_DIM = 64
OUT_DIM = 128
ISS_PAD = 2048
ISS_BS = 512
CAT_PAD = 128

f32 = jnp.float32
bf16 = jnp.bfloat16

_NC, _NS = 2, 16
_NW = _NC * _NS
_BPW = B // _NW

_sc_mesh = plsc.VectorSubcoreMesh(core_axis_name="c", subcore_axis_name="s")


@functools.partial(
    pl.kernel,
    out_type=(jax.ShapeDtypeStruct((B,), jnp.int32),
              jax.ShapeDtypeStruct((B,), jnp.int32)),
    mesh=_sc_mesh,
    scratch_types=[pltpu.VMEM((_BPW,), jnp.int32),
                   pltpu.VMEM((_BPW,), jnp.int32),
                   pltpu.VMEM((_BPW,), jnp.int32),
                   pltpu.SemaphoreType.DMA],
    compiler_params=pltpu.CompilerParams(use_tc_tiling_on_sc=False),
)
def _sc_gather(ids_hbm, iss_hbm, sec_hbm, iss_out, sec_out,
               idx_v, iss_v, sec_v, sem):
    wid = lax.axis_index("s") * _NC + lax.axis_index("c")
    base = wid * _BPW
    pltpu.sync_copy(ids_hbm.at[pl.ds(base, _BPW)], idx_v)
    c2 = pltpu.async_copy(iss_hbm.at[idx_v], iss_v, sem)
    c3 = pltpu.async_copy(sec_hbm.at[idx_v], sec_v, sem)
    c2.wait()
    c3.wait()
    pltpu.sync_copy(iss_v, iss_out.at[pl.ds(base, _BPW)])
    pltpu.sync_copy(sec_v, sec_out.at[pl.ds(base, _BPW)])


def _dense_a_body(iss_ref, sec_ref, catr_ref, catc_ref, nums_ref,
                  ert_ref, ecty_ref, nW1_ref, nb1_ref, nW2_ref, nb2_ref,
                  A_ref, ab1_ref, pre_ref, invi_ref, invs_ref):
    iota_c = jax.lax.broadcasted_iota(jnp.int32, (B, CAT_PAD), 1)
    ones_h = jnp.full((B, 1), 1.0, bf16)

    R = (catr_ref[:] == iota_c).astype(f32)
    e_rat = jnp.dot(R, ert_ref[:], preferred_element_type=f32)
    C = (catc_ref[:] == iota_c).astype(f32)
    e_cty = jnp.dot(C, ecty_ref[:], preferred_element_type=f32)

    h1 = jnp.maximum(jnp.dot(nums_ref[:], nW1_ref[:], preferred_element_type=f32)
                     + nb1_ref[:], 0.0)
    h_num = jnp.maximum(jnp.dot(h1, nW2_ref[:], preferred_element_type=f32)
                        + nb2_ref[:], 0.0)

    A_h = A_ref[:].astype(bf16)
    Ya = (jnp.dot(e_rat.astype(bf16), A_h[64:80, :], preferred_element_type=f32)
          + jnp.dot(e_cty.astype(bf16), A_h[80:96, :], preferred_element_type=f32)
          + jnp.dot(h_num.astype(bf16), A_h[96:160, :], preferred_element_type=f32))
    Y1a = Ya[:, 0:128]
    Y2a = Ya[:, 128:256]
    Y3a = Ya[:, 256:384]

    S = (sec_ref[:] == iota_c).astype(bf16)
    sec_cnt = jax.lax.dot_general(S, ones_h, (((0,), (0,)), ((), ())),
                                  preferred_element_type=f32)
    invs = 1.0 / jnp.maximum(sec_cnt, 1.0)
    invs_ref[:] = invs
    sec_sums = jax.lax.dot_general(S, Y3a.astype(bf16), (((0,), (0,)), ((), ())),
                                   preferred_element_type=f32)
    h_sec = jnp.dot(S, (sec_sums * invs).astype(bf16),
                    preferred_element_type=f32)

    Y2h = Y2a.astype(bf16)
    h_iss = jnp.zeros((B, OUT_DIM), f32)
    for k in range(ISS_PAD // ISS_BS):
        iota_k = jax.lax.broadcasted_iota(jnp.int32, (B, ISS_BS), 1) + k * ISS_BS
        Sk = (iss_ref[:] == iota_k).astype(bf16)
        cnt_k = jax.lax.dot_general(Sk, ones_h, (((0,), (0,)), ((), ())),
                                    preferred_element_type=f32)
        invi_k = 1.0 / jnp.maximum(cnt_k, 1.0)
        invi_ref[k * ISS_BS:(k + 1) * ISS_BS, :] = invi_k
        sums_k = jax.lax.dot_general(Sk, Y2h, (((0,), (0,)), ((), ())),
                                     preferred_element_type=f32)
        h_iss = h_iss + jnp.dot(Sk, (sums_k * invi_k).astype(bf16),
                                preferred_element_type=f32)
    pre_ref[:] = Y1a + h_iss + h_sec + ab1_ref[:]


def _dense_a(issuers, sectors, cat_rating, cat_country, nums,
             ert_pad, ecty_pad, nW1, nb1, nW2, nb2, A_comb, ab1):
    return pl.pallas_call(
        _dense_a_body,
        out_shape=(jax.ShapeDtypeStruct((B, OUT_DIM), f32),
                   jax.ShapeDtypeStruct((ISS_PAD, 1), f32),
                   jax.ShapeDtypeStruct((CAT_PAD, 1), f32)),
    )(issuers, sectors, cat_rating, cat_country, nums,
      ert_pad, ecty_pad, nW1, nb1, nW2, nb2, A_comb, ab1)


def _dense_b_body(eid_ref, iss_ref, sec_ref, pre_ref, invi_ref, invs_ref,
                  A_ref, aW2_ref, ab2_ref, out_ref):
    iota_c = jax.lax.broadcasted_iota(jnp.int32, (B, CAT_PAD), 1)
    eid_h = eid_ref[:].astype(bf16)
    A_h = A_ref[:].astype(bf16)
    A1id = A_h[0:64, 0:128]
    A2id = A_h[0:64, 128:256]
    A3id = A_h[0:64, 256:384]

    Yb1 = jnp.dot(eid_h, A1id, preferred_element_type=f32)

    pool_i = jnp.zeros((B, NODE_ID_DIM), f32)
    for k in range(ISS_PAD // ISS_BS):
        iota_k = jax.lax.broadcasted_iota(jnp.int32, (B, ISS_BS), 1) + k * ISS_BS
        Sk = (iss_ref[:] == iota_k).astype(bf16)
        sums_k = jax.lax.dot_general(Sk, eid_h, (((0,), (0,)), ((), ())),
                                     preferred_element_type=f32)
        means_k = sums_k * invi_ref[k * ISS_BS:(k + 1) * ISS_BS, :]
        pool_i = pool_i + jnp.dot(Sk, means_k.astype(bf16),
                                  preferred_element_type=f32)
    c_iss = jnp.dot(pool_i.astype(bf16), A2id, preferred_element_type=f32)

    S = (sec_ref[:] == iota_c).astype(bf16)
    sums_s = jax.lax.dot_general(S, eid_h, (((0,), (0,)), ((), ())),
                                 preferred_element_type=f32)
    means_s = sums_s * invs_ref[:]
    pool_s = jnp.dot(S, means_s.astype(bf16), preferred_element_type=f32)
    c_sec = jnp.dot(pool_s.astype(bf16), A3id, preferred_element_type=f32)

    pre = jnp.maximum(pre_ref[:] + Yb1 + c_iss + c_sec, 0.0)
    out_ref[:] = (jnp.dot(pre.astype(bf16), aW2_ref[:].astype(bf16),
                          preferred_element_type=f32) + ab2_ref[:])


def _dense_b(e_id, issuers, sectors, pre_a, invi, invs, A_comb, aW2, ab2):
    return pl.pallas_call(
        _dense_b_body,
        out_shape=jax.ShapeDtypeStruct((B, OUT_DIM), f32),
    )(e_id, issuers, sectors, pre_a, invi, invs, A_comb, aW2, ab2)


def kernel(node_ids, cat_rating, cat_country, nums, node_to_issuer, node_to_sector,
           id_emb, emb_rating, emb_country, nW1, nb1, nW2, nb2, aW1, ab1, aW2, ab2):
    issuers, sectors = _sc_gather(
        node_ids.astype(jnp.int32),
        node_to_issuer.astype(jnp.int32),
        node_to_sector.astype(jnp.int32))
    e_id = jnp.take(id_emb, node_ids, axis=0)

    ert_pad = jnp.zeros((CAT_PAD, 16), f32).at[:emb_rating.shape[0]].set(emb_rating)
    ecty_pad = jnp.zeros((CAT_PAD, 16), f32).at[:emb_country.shape[0]].set(emb_country)
    A_comb = jnp.concatenate([aW1[0:160], aW1[160:320], aW1[320:480]], axis=1)

    iss2 = issuers.reshape(B, 1)
    sec2 = sectors.reshape(B, 1)
    pre_a, invi, invs = _dense_a(
        iss2, sec2,
        cat_rating.reshape(B, 1).astype(jnp.int32),
        cat_country.reshape(B, 1).astype(jnp.int32),
        nums, ert_pad, ecty_pad,
        nW1, nb1.reshape(1, -1), nW2, nb2.reshape(1, -1),
        A_comb, ab1.reshape(1, -1))
    return _dense_b(e_id, iss2, sec2, pre_a, invi, invs,
                    A_comb, aW2, ab2.reshape(1, -1))


# speedup vs baseline: 1.3013x; 1.3013x over previous
import functools

import jax
import jax.numpy as jnp
from jax import lax
from jax.experimental import pallas as pl
from jax.experimental.pallas import tpu as pltpu
from jax.experimental.pallas import tpu_sc as plsc

B = 4096
NODE_ID_DIM = 64
OUT_DIM = 128
ISS_PAD = 2048
ISS_BS = 1024
CAT_PAD = 128


_NC, _NS = 2, 16
_NW = _NC * _NS
_BPW = B // _NW

_sc_mesh = plsc.VectorSubcoreMesh(core_axis_name="c", subcore_axis_name="s")


@functools.partial(
    pl.kernel,
    out_type=(jax.ShapeDtypeStruct((B,), jnp.int32),
              jax.ShapeDtypeStruct((B,), jnp.int32)),
    mesh=_sc_mesh,
    scratch_types=[pltpu.VMEM((_BPW,), jnp.int32),
                   pltpu.VMEM((_BPW,), jnp.int32),
                   pltpu.VMEM((_BPW,), jnp.int32),
                   pltpu.SemaphoreType.DMA],
    compiler_params=pltpu.CompilerParams(use_tc_tiling_on_sc=False),
)
def _sc_gather(ids_hbm, iss_hbm, sec_hbm, iss_out, sec_out,
               idx_v, iss_v, sec_v, sem):
    wid = lax.axis_index("s") * _NC + lax.axis_index("c")
    base = wid * _BPW
    pltpu.sync_copy(ids_hbm.at[pl.ds(base, _BPW)], idx_v)
    c2 = pltpu.async_copy(iss_hbm.at[idx_v], iss_v, sem)
    c3 = pltpu.async_copy(sec_hbm.at[idx_v], sec_v, sem)
    c2.wait()
    c3.wait()
    pltpu.sync_copy(iss_v, iss_out.at[pl.ds(base, _BPW)])
    pltpu.sync_copy(sec_v, sec_out.at[pl.ds(base, _BPW)])


def _dense_body(eid_ref, iss_ref, sec_ref, catr_ref, catc_ref, nums_ref,
                ert_ref, ecty_ref, nW1_ref, nb1_ref, nW2_ref, nb2_ref,
                A_ref, ab1_ref, aW2_ref, ab2_ref, out_ref):
    f32 = jnp.float32
    iota_c = jax.lax.broadcasted_iota(jnp.int32, (B, CAT_PAD), 1)
    ones_col = jnp.full((B, 1), 1.0, f32)

    R = (catr_ref[:] == iota_c).astype(f32)
    e_rat = jnp.dot(R, ert_ref[:], preferred_element_type=f32)
    C = (catc_ref[:] == iota_c).astype(f32)
    e_cty = jnp.dot(C, ecty_ref[:], preferred_element_type=f32)

    h1 = jnp.maximum(jnp.dot(nums_ref[:], nW1_ref[:], preferred_element_type=f32)
                     + nb1_ref[:], 0.0)
    h_num = jnp.maximum(jnp.dot(h1, nW2_ref[:], preferred_element_type=f32)
                        + nb2_ref[:], 0.0)

    bf16 = jnp.bfloat16
    A_h = A_ref[:].astype(bf16)
    Y = (jnp.dot(eid_ref[:].astype(bf16), A_h[0:64, :], preferred_element_type=f32)
         + jnp.dot(e_rat.astype(bf16), A_h[64:80, :], preferred_element_type=f32)
         + jnp.dot(e_cty.astype(bf16), A_h[80:96, :], preferred_element_type=f32)
         + jnp.dot(h_num.astype(bf16), A_h[96:160, :], preferred_element_type=f32))
    Y1 = Y[:, 0:128]
    Y2 = Y[:, 128:256]
    Y3 = Y[:, 256:384]

    S = (sec_ref[:] == iota_c).astype(bf16)
    sec_sums = jax.lax.dot_general(S, Y3.astype(bf16), (((0,), (0,)), ((), ())),
                                   preferred_element_type=f32)
    sec_cnt = jax.lax.dot_general(S, ones_col.astype(bf16), (((0,), (0,)), ((), ())),
                                  preferred_element_type=f32)
    sec_means = sec_sums / jnp.maximum(sec_cnt, 1.0)
    h_sec = jnp.dot(S, sec_means.astype(bf16), preferred_element_type=f32)

    Y2h = Y2.astype(bf16)
    h_iss = jnp.zeros((B, OUT_DIM), f32)
    for k in range(ISS_PAD // ISS_BS):
        iota_k = jax.lax.broadcasted_iota(jnp.int32, (B, ISS_BS), 1) + k * ISS_BS
        Sk = (iss_ref[:] == iota_k).astype(bf16)
        sums_k = jax.lax.dot_general(Sk, Y2h, (((0,), (0,)), ((), ())),
                                     preferred_element_type=f32)
        cnt_k = jax.lax.dot_general(Sk, ones_col.astype(bf16), (((0,), (0,)), ((), ())),
                                    preferred_element_type=f32)
        means_k = sums_k / jnp.maximum(cnt_k, 1.0)
        h_iss = h_iss + jnp.dot(Sk, means_k.astype(bf16), preferred_element_type=f32)

    pre = jnp.maximum(Y1 + h_iss + h_sec + ab1_ref[:], 0.0)
    out_ref[:] = (jnp.dot(pre.astype(bf16), aW2_ref[:].astype(bf16),
                          preferred_element_type=f32) + ab2_ref[:])


def _dense_call(e_id, issuers, sectors, cat_rating, cat_country, nums,
                ert_pad, ecty_pad, nW1, nb1, nW2, nb2, A_comb, ab1, aW2, ab2):
    return pl.pallas_call(
        _dense_body,
        out_shape=jax.ShapeDtypeStruct((B, OUT_DIM), jnp.float32),
    )(e_id, issuers, sectors, cat_rating, cat_country, nums,
      ert_pad, ecty_pad, nW1, nb1, nW2, nb2, A_comb, ab1, aW2, ab2)


def kernel(node_ids, cat_rating, cat_country, nums, node_to_issuer, node_to_sector,
           id_emb, emb_rating, emb_country, nW1, nb1, nW2, nb2, aW1, ab1, aW2, ab2):
    issuers, sectors = _sc_gather(
        node_ids.astype(jnp.int32),
        node_to_issuer.astype(jnp.int32),
        node_to_sector.astype(jnp.int32))
    e_id = jnp.take(id_emb, node_ids, axis=0)

    ert_pad = jnp.zeros((CAT_PAD, 16), jnp.float32).at[:emb_rating.shape[0]].set(emb_rating)
    ecty_pad = jnp.zeros((CAT_PAD, 16), jnp.float32).at[:emb_country.shape[0]].set(emb_country)
    A_comb = jnp.concatenate([aW1[0:160], aW1[160:320], aW1[320:480]], axis=1)

    return _dense_call(
        e_id,
        issuers.reshape(B, 1),
        sectors.reshape(B, 1),
        cat_rating.reshape(B, 1).astype(jnp.int32),
        cat_country.reshape(B, 1).astype(jnp.int32),
        nums,
        ert_pad, ecty_pad,
        nW1, nb1.reshape(1, -1), nW2, nb2.reshape(1, -1),
        A_comb, ab1.reshape(1, -1), aW2, ab2.reshape(1, -1),
    )


# speedup vs baseline: 1.3065x; 1.0040x over previous
import functools

import jax
import jax.numpy as jnp
from jax import lax
from jax.experimental import pallas as pl
from jax.experimental.pallas import tpu as pltpu
from jax.experimental.pallas import tpu_sc as plsc

B = 4096
NODE_ID_DIM = 64
OUT_DIM = 128
ISS_PAD = 2048
ISS_BS = 2048
CAT_PAD = 128


_NC, _NS = 2, 16
_NW = _NC * _NS
_BPW = B // _NW

_sc_mesh = plsc.VectorSubcoreMesh(core_axis_name="c", subcore_axis_name="s")


@functools.partial(
    pl.kernel,
    out_type=(jax.ShapeDtypeStruct((B,), jnp.int32),
              jax.ShapeDtypeStruct((B,), jnp.int32)),
    mesh=_sc_mesh,
    scratch_types=[pltpu.VMEM((_BPW,), jnp.int32),
                   pltpu.VMEM((_BPW,), jnp.int32),
                   pltpu.VMEM((_BPW,), jnp.int32),
                   pltpu.SemaphoreType.DMA],
    compiler_params=pltpu.CompilerParams(use_tc_tiling_on_sc=False),
)
def _sc_gather(ids_hbm, iss_hbm, sec_hbm, iss_out, sec_out,
               idx_v, iss_v, sec_v, sem):
    wid = lax.axis_index("s") * _NC + lax.axis_index("c")
    base = wid * _BPW
    pltpu.sync_copy(ids_hbm.at[pl.ds(base, _BPW)], idx_v)
    c2 = pltpu.async_copy(iss_hbm.at[idx_v], iss_v, sem)
    c3 = pltpu.async_copy(sec_hbm.at[idx_v], sec_v, sem)
    c2.wait()
    c3.wait()
    pltpu.sync_copy(iss_v, iss_out.at[pl.ds(base, _BPW)])
    pltpu.sync_copy(sec_v, sec_out.at[pl.ds(base, _BPW)])


def _dense_body(eid_ref, iss_ref, sec_ref, catr_ref, catc_ref, nums_ref,
                ert_ref, ecty_ref, nW1_ref, nb1_ref, nW2_ref, nb2_ref,
                A_ref, ab1_ref, aW2_ref, ab2_ref, out_ref):
    f32 = jnp.float32
    iota_c = jax.lax.broadcasted_iota(jnp.int32, (B, CAT_PAD), 1)
    ones_col = jnp.full((B, 1), 1.0, f32)

    R = (catr_ref[:] == iota_c).astype(f32)
    e_rat = jnp.dot(R, ert_ref[:], preferred_element_type=f32)
    C = (catc_ref[:] == iota_c).astype(f32)
    e_cty = jnp.dot(C, ecty_ref[:], preferred_element_type=f32)

    h1 = jnp.maximum(jnp.dot(nums_ref[:], nW1_ref[:], preferred_element_type=f32)
                     + nb1_ref[:], 0.0)
    h_num = jnp.maximum(jnp.dot(h1, nW2_ref[:], preferred_element_type=f32)
                        + nb2_ref[:], 0.0)

    bf16 = jnp.bfloat16
    A_h = A_ref[:].astype(bf16)
    Y = (jnp.dot(eid_ref[:].astype(bf16), A_h[0:64, :], preferred_element_type=f32)
         + jnp.dot(e_rat.astype(bf16), A_h[64:80, :], preferred_element_type=f32)
         + jnp.dot(e_cty.astype(bf16), A_h[80:96, :], preferred_element_type=f32)
         + jnp.dot(h_num.astype(bf16), A_h[96:160, :], preferred_element_type=f32))
    Y1 = Y[:, 0:128]
    Y2 = Y[:, 128:256]
    Y3 = Y[:, 256:384]

    S = (sec_ref[:] == iota_c).astype(bf16)
    sec_sums = jax.lax.dot_general(S, Y3.astype(bf16), (((0,), (0,)), ((), ())),
                                   preferred_element_type=f32)
    sec_cnt = jax.lax.dot_general(S, ones_col.astype(bf16), (((0,), (0,)), ((), ())),
                                  preferred_element_type=f32)
    sec_means = sec_sums / jnp.maximum(sec_cnt, 1.0)
    h_sec = jnp.dot(S, sec_means.astype(bf16), preferred_element_type=f32)

    Y2h = Y2.astype(bf16)
    h_iss = jnp.zeros((B, OUT_DIM), f32)
    for k in range(ISS_PAD // ISS_BS):
        iota_k = jax.lax.broadcasted_iota(jnp.int32, (B, ISS_BS), 1) + k * ISS_BS
        Sk = (iss_ref[:] == iota_k).astype(bf16)
        sums_k = jax.lax.dot_general(Sk, Y2h, (((0,), (0,)), ((), ())),
                                     preferred_element_type=f32)
        cnt_k = jax.lax.dot_general(Sk, ones_col.astype(bf16), (((0,), (0,)), ((), ())),
                                    preferred_element_type=f32)
        means_k = sums_k / jnp.maximum(cnt_k, 1.0)
        h_iss = h_iss + jnp.dot(Sk, means_k.astype(bf16), preferred_element_type=f32)

    pre = jnp.maximum(Y1 + h_iss + h_sec + ab1_ref[:], 0.0)
    out_ref[:] = (jnp.dot(pre.astype(bf16), aW2_ref[:].astype(bf16),
                          preferred_element_type=f32) + ab2_ref[:])


def _dense_call(e_id, issuers, sectors, cat_rating, cat_country, nums,
                ert_pad, ecty_pad, nW1, nb1, nW2, nb2, A_comb, ab1, aW2, ab2):
    return pl.pallas_call(
        _dense_body,
        out_shape=jax.ShapeDtypeStruct((B, OUT_DIM), jnp.float32),
    )(e_id, issuers, sectors, cat_rating, cat_country, nums,
      ert_pad, ecty_pad, nW1, nb1, nW2, nb2, A_comb, ab1, aW2, ab2)


def kernel(node_ids, cat_rating, cat_country, nums, node_to_issuer, node_to_sector,
           id_emb, emb_rating, emb_country, nW1, nb1, nW2, nb2, aW1, ab1, aW2, ab2):
    issuers, sectors = _sc_gather(
        node_ids.astype(jnp.int32),
        node_to_issuer.astype(jnp.int32),
        node_to_sector.astype(jnp.int32))
    e_id = jnp.take(id_emb, node_ids, axis=0)

    ert_pad = jnp.zeros((CAT_PAD, 16), jnp.float32).at[:emb_rating.shape[0]].set(emb_rating)
    ecty_pad = jnp.zeros((CAT_PAD, 16), jnp.float32).at[:emb_country.shape[0]].set(emb_country)
    A_comb = jnp.concatenate([aW1[0:160], aW1[160:320], aW1[320:480]], axis=1)

    return _dense_call(
        e_id,
        issuers.reshape(B, 1),
        sectors.reshape(B, 1),
        cat_rating.reshape(B, 1).astype(jnp.int32),
        cat_country.reshape(B, 1).astype(jnp.int32),
        nums,
        ert_pad, ecty_pad,
        nW1, nb1.reshape(1, -1), nW2, nb2.reshape(1, -1),
        A_comb, ab1.reshape(1, -1), aW2, ab2.reshape(1, -1),
    )
